# baseline (speedup 1.0000x reference)
import functools

import jax
import jax.numpy as jnp
from jax import lax
from jax.experimental import pallas as pl
from jax.experimental.pallas import tpu as pltpu
from jax.experimental.pallas import tpu_sc as plsc

NC = 2
NS = 16
NW = NC * NS
LANES = 16
CHUNK = 128


def _sc_mesh():
    return plsc.VectorSubcoreMesh(core_axis_name="c", subcore_axis_name="s")


_SC_PARAMS = pltpu.CompilerParams(use_tc_tiling_on_sc=False)


DEGW = LANES


def _sc_degree(dst2d, zeros_tile, ones_chunk, n_acc):
    n_chunks = dst2d.shape[0]
    cpw = n_chunks // NW
    tail = n_chunks - cpw * NW
    orows = n_acc // NS
    wave = cpw
    for w in (32, 26, 24, 16, 13, 8):
        if cpw % w == 0:
            wave = w
            break
    nwaves = cpw // wave

    @functools.partial(
        pl.kernel,
        out_type=jax.ShapeDtypeStruct((NC, n_acc, DEGW), jnp.float32),
        mesh=_sc_mesh(),
        compiler_params=_SC_PARAMS,
        scratch_types=[
            pltpu.VMEM((cpw, CHUNK), jnp.int32),
            pltpu.VMEM((CHUNK, DEGW), jnp.float32),
            pltpu.VMEM((orows, DEGW), jnp.float32),
            pltpu.VMEM_SHARED((n_acc, DEGW), jnp.float32),
            pltpu.SemaphoreType.DMA((4,)),
        ],
    )
    def k(dst_hbm, z_hbm, ones_hbm, out_hbm, dstv, ones_v, stage, acc, sems):
        cid = lax.axis_index("c")
        sid = lax.axis_index("s")
        wid = cid * NS + sid
        ssem = sems.at[3]

        cz = pltpu.async_copy(z_hbm, stage, sems.at[0])
        co = pltpu.async_copy(ones_hbm, ones_v, sems.at[1])
        cd = pltpu.async_copy(dst_hbm.at[pl.ds(wid * cpw, cpw)], dstv,
                              sems.at[2])
        cz.wait()
        pltpu.sync_copy(stage, acc.at[pl.ds(sid * orows, orows)])
        co.wait()
        cd.wait()
        plsc.subcore_barrier()

        @pl.loop(0, nwaves)
        def _(j):
            base = j * wave

            @pl.loop(0, wave)
            def _(i):
                pltpu.async_copy(ones_v, acc.at[dstv.at[base + i]], ssem,
                                 add=True)

            @pl.loop(0, wave)
            def _(i):
                pltpu.make_async_copy(
                    ones_v, acc.at[dstv.at[base + i]], ssem).wait()

        if tail:
            @pl.when(wid >= NW - tail)
            def _():
                tcid = n_chunks - NW + wid
                pltpu.sync_copy(dst_hbm.at[tcid], dstv.at[0])
                pltpu.sync_copy(ones_v, acc.at[dstv.at[0]], add=True)

        plsc.subcore_barrier()
        pltpu.sync_copy(acc.at[pl.ds(sid * orows, orows)],
                        out_hbm.at[cid, pl.ds(sid * orows, orows)])

    return k(dst2d, zeros_tile, ones_chunk)


def _sc_gather_scatter(g, src2d, dst2d, zeros_tile, n_acc):
    n_chunks = src2d.shape[0]
    cpw = n_chunks // NW
    tail = n_chunks - cpw * NW
    orows = n_acc // NS
    nb = 1
    for cand in (16, 13, 12, 8, 6, 4, 2):
        if cpw % cand == 0:
            nb = cand
            break
    nring = cpw // nb

    @functools.partial(
        pl.kernel,
        out_type=jax.ShapeDtypeStruct((NC, n_acc, LANES), jnp.float32),
        mesh=_sc_mesh(),
        compiler_params=_SC_PARAMS,
        scratch_types=[
            pltpu.VMEM((cpw, CHUNK), jnp.int32),
            pltpu.VMEM((cpw, CHUNK), jnp.int32),
            [pltpu.VMEM((CHUNK, LANES), jnp.float32) for _ in range(nb)],
            pltpu.VMEM((orows, LANES), jnp.float32),
            pltpu.VMEM_SHARED((n_acc, LANES), jnp.float32),
            pltpu.SemaphoreType.DMA((nb,)),
            pltpu.SemaphoreType.DMA((nb,)),
        ],
    )
    def k(g_hbm, src_hbm, dst_hbm, z_hbm, out_hbm, srcv, dstv, rows, stage,
          acc, gsem, ssem):
        cid = lax.axis_index("c")
        sid = lax.axis_index("s")
        wid = cid * NS + sid

        cz = pltpu.async_copy(z_hbm, stage, ssem.at[0])
        cs = pltpu.async_copy(src_hbm.at[pl.ds(wid * cpw, cpw)], srcv,
                              gsem.at[0])
        cd = pltpu.async_copy(dst_hbm.at[pl.ds(wid * cpw, cpw)], dstv,
                              ssem.at[1])
        cs.wait()
        for b in range(nb):
            pltpu.async_copy(g_hbm.at[srcv.at[b]], rows[b], gsem.at[b])
        cz.wait()
        pltpu.sync_copy(stage, acc.at[pl.ds(sid * orows, orows)])
        cd.wait()
        plsc.subcore_barrier()

        @pl.loop(0, nring)
        def _(j):
            base = j * nb
            for b in range(nb):
                i = base + b
                pltpu.make_async_copy(
                    g_hbm.at[srcv.at[i]], rows[b], gsem.at[b]).wait()
                pltpu.async_copy(rows[b], acc.at[dstv.at[i]], ssem.at[b],
                                 add=True)

            @pl.when(j < nring - 1)
            def _():
                for b in range(nb):
                    i = base + b
                    pltpu.make_async_copy(
                        rows[b], acc.at[dstv.at[i]], ssem.at[b]).wait()
                    pltpu.async_copy(
                        g_hbm.at[srcv.at[i + nb]], rows[b], gsem.at[b])

        for b in range(nb):
            i = (nring - 1) * nb + b
            pltpu.make_async_copy(
                rows[b], acc.at[dstv.at[i]], ssem.at[b]).wait()

        if tail:
            @pl.when(wid >= NW - tail)
            def _():
                tcid = n_chunks - NW + wid
                pltpu.sync_copy(src_hbm.at[tcid], srcv.at[0])
                pltpu.sync_copy(dst_hbm.at[tcid], dstv.at[0])
                pltpu.sync_copy(g_hbm.at[srcv.at[0]], rows[0])
                pltpu.sync_copy(rows[0], acc.at[dstv.at[0]], add=True)

        plsc.subcore_barrier()
        pltpu.sync_copy(acc.at[pl.ds(sid * orows, orows)],
                        out_hbm.at[cid, pl.ds(sid * orows, orows)])

    return k(g, src2d, dst2d, zeros_tile)


def _tc_call(body, out_shapes, *args):
    return pl.pallas_call(
        body,
        out_shape=[jax.ShapeDtypeStruct(s, jnp.float32) for s in out_shapes],
    )(*args)


def _mm_body(x_ref, w_ref, o_ref):
    o_ref[...] = jnp.dot(x_ref[...], w_ref[...],
                         preferred_element_type=jnp.float32)


def _norm_body(degp_ref, h_ref, b_ref, g_ref, disb_ref, a_ref):
    m = h_ref.shape[0]
    degp = degp_ref[...]
    deg = degp[0, :m] + degp[1, :m] + 1.0
    disb = 1.0 / jnp.sqrt(deg)
    h = h_ref[...]
    g_ref[...] = disb * h
    disb_ref[...] = disb
    a_ref[...] = disb * disb * h + b_ref[...]


def _mid_body(sp_ref, disb_ref, a_ref, w2_ref, b2_ref, g2_ref, a2_ref):
    m = disb_ref.shape[0]
    disb = disb_ref[...]
    sp = sp_ref[...]
    out1 = jnp.maximum(disb * (sp[0, :m] + sp[1, :m]) + a_ref[...], 0.0)
    h2 = jnp.dot(out1, w2_ref[...], preferred_element_type=jnp.float32)
    g2_ref[...] = disb * h2
    a2_ref[...] = disb * disb * h2 + b2_ref[...]


def _final_body(sp_ref, disb_ref, a2_ref, o_ref):
    m = disb_ref.shape[0]
    sp = sp_ref[...]
    out2 = disb_ref[...] * (sp[0, :m] + sp[1, :m]) + a2_ref[...]
    cols = []
    for j in range(8):
        v = out2[:, 16 * j:16 * (j + 1)]
        z = v - jnp.max(v, axis=1, keepdims=True)
        cols.append(z - jnp.log(jnp.sum(jnp.exp(z), axis=1, keepdims=True)))
    o_ref[...] = jnp.concatenate(cols, axis=1)


@jax.jit
def _gcn(x, edge_index, W1, b1, W2, b2):
    n = x.shape[0]
    e = edge_index.shape[1]

    n_acc = -(-n // (NS * 8)) * NS * 8
    mp, ma = n // 8, n_acc // 8
    src2d = edge_index[0].reshape(e // CHUNK, CHUNK)
    dst2d = edge_index[1].reshape(e // CHUNK, CHUNK)
    zeros_tile = jnp.zeros((n_acc // NS, LANES), jnp.float32)
    ones_chunk = jnp.ones((CHUNK, LANES), jnp.float32)
    b1p = jnp.tile(b1, 8).reshape(1, 128)
    b2p = jnp.tile(b2, 8).reshape(1, 128)
    eye8 = jnp.eye(8, dtype=jnp.float32)
    w1blk = jnp.reshape(
        eye8[:, None, :, None] * W1[None, :, None, :], (8 * 128, 128))
    w2blk = jnp.reshape(
        eye8[:, None, :, None] * W2[None, :, None, :], (128, 128))

    degp = _sc_degree(dst2d, zeros_tile, ones_chunk, n_acc)
    (h1p,) = _tc_call(_mm_body, [(mp, 128)], x.reshape(mp, 8 * 128), w1blk)
    g1p, disbp, a1p = _tc_call(_norm_body, [(mp, 128)] * 3,
                               degp.reshape(NC, ma, 128), h1p, b1p)
    s1p = _sc_gather_scatter(g1p.reshape(n, LANES), src2d, dst2d,
                             zeros_tile, n_acc)
    g2p, a2p = _tc_call(_mid_body, [(mp, 128)] * 2,
                        s1p.reshape(NC, ma, 128), disbp, a1p, w2blk, b2p)
    s2p = _sc_gather_scatter(g2p.reshape(n, LANES), src2d, dst2d,
                             zeros_tile, n_acc)
    (outp,) = _tc_call(_final_body, [(mp, 128)],
                       s2p.reshape(NC, ma, 128), disbp, a2p)
    return outp.reshape(n, LANES)


def kernel(x, edge_index, W1, b1, W2, b2):
    return _gcn(x, edge_index, W1, b1, W2, b2)


# speedup vs baseline: 1.1198x; 1.1198x over previous
import functools

import jax
import jax.numpy as jnp
from jax import lax
from jax.experimental import pallas as pl
from jax.experimental.pallas import tpu as pltpu
from jax.experimental.pallas import tpu_sc as plsc

NC = 2
NS = 16
NW = NC * NS
LANES = 16
CHUNK = 128


def _sc_mesh():
    return plsc.VectorSubcoreMesh(core_axis_name="c", subcore_axis_name="s")


_SC_PARAMS = pltpu.CompilerParams(use_tc_tiling_on_sc=False)


DEGW = LANES


def _sc_degree(edge3d, zeros_tile, ones_chunk, n_acc):
    n_chunks = edge3d.shape[1]
    cpw = n_chunks // NW
    tail = n_chunks - cpw * NW
    orows = n_acc // NS
    wave = cpw
    for w in (32, 26, 24, 16, 13, 8):
        if cpw % w == 0:
            wave = w
            break
    nwaves = cpw // wave

    @functools.partial(
        pl.kernel,
        out_type=jax.ShapeDtypeStruct((NC, n_acc, DEGW), jnp.float32),
        mesh=_sc_mesh(),
        compiler_params=_SC_PARAMS,
        scratch_types=[
            pltpu.VMEM((cpw, CHUNK), jnp.int32),
            pltpu.VMEM((CHUNK, DEGW), jnp.float32),
            pltpu.VMEM((orows, DEGW), jnp.float32),
            pltpu.VMEM_SHARED((n_acc, DEGW), jnp.float32),
            pltpu.SemaphoreType.DMA((4,)),
        ],
    )
    def k(e_hbm, z_hbm, ones_hbm, out_hbm, dstv, ones_v, stage, acc, sems):
        cid = lax.axis_index("c")
        sid = lax.axis_index("s")
        wid = cid * NS + sid
        ssem = sems.at[3]

        cz = pltpu.async_copy(z_hbm, stage, sems.at[0])
        co = pltpu.async_copy(ones_hbm, ones_v, sems.at[1])
        cd = pltpu.async_copy(e_hbm.at[1, pl.ds(wid * cpw, cpw)], dstv,
                              sems.at[2])
        cz.wait()
        pltpu.sync_copy(stage, acc.at[pl.ds(sid * orows, orows)])
        co.wait()
        cd.wait()
        plsc.subcore_barrier()

        @pl.loop(0, nwaves)
        def _(j):
            base = j * wave

            @pl.loop(0, wave)
            def _(i):
                pltpu.async_copy(ones_v, acc.at[dstv.at[base + i]], ssem,
                                 add=True)

            @pl.loop(0, wave)
            def _(i):
                pltpu.make_async_copy(
                    ones_v, acc.at[dstv.at[base + i]], ssem).wait()

        if tail:
            @pl.when(wid >= NW - tail)
            def _():
                tcid = n_chunks - NW + wid
                pltpu.sync_copy(e_hbm.at[1, tcid], dstv.at[0])
                pltpu.sync_copy(ones_v, acc.at[dstv.at[0]], add=True)

        plsc.subcore_barrier()
        pltpu.sync_copy(acc.at[pl.ds(sid * orows, orows)],
                        out_hbm.at[cid, pl.ds(sid * orows, orows)])

    return k(edge3d, zeros_tile, ones_chunk)


def _sc_gather_scatter(g, edge3d, zeros_tile, n_acc):
    n_chunks = edge3d.shape[1]
    cpw = n_chunks // NW
    tail = n_chunks - cpw * NW
    orows = n_acc // NS
    nb = 1
    for cand in (16, 13, 12, 8, 6, 4, 2):
        if cpw % cand == 0:
            nb = cand
            break
    nring = cpw // nb

    @functools.partial(
        pl.kernel,
        out_type=jax.ShapeDtypeStruct((NC, n_acc, LANES), jnp.float32),
        mesh=_sc_mesh(),
        compiler_params=_SC_PARAMS,
        scratch_types=[
            pltpu.VMEM((cpw, CHUNK), jnp.int32),
            pltpu.VMEM((cpw, CHUNK), jnp.int32),
            [pltpu.VMEM((CHUNK, LANES), jnp.float32) for _ in range(nb)],
            pltpu.VMEM((orows, LANES), jnp.float32),
            pltpu.VMEM_SHARED((n_acc, LANES), jnp.float32),
            pltpu.SemaphoreType.DMA((nb,)),
            pltpu.SemaphoreType.DMA((nb,)),
        ],
    )
    def k(g_hbm, e_hbm, z_hbm, out_hbm, srcv, dstv, rows, stage,
          acc, gsem, ssem):
        cid = lax.axis_index("c")
        sid = lax.axis_index("s")
        wid = cid * NS + sid

        cz = pltpu.async_copy(z_hbm, stage, ssem.at[0])
        cs = pltpu.async_copy(e_hbm.at[0, pl.ds(wid * cpw, cpw)], srcv,
                              gsem.at[0])
        cd = pltpu.async_copy(e_hbm.at[1, pl.ds(wid * cpw, cpw)], dstv,
                              ssem.at[1])
        cs.wait()
        for b in range(nb):
            pltpu.async_copy(g_hbm.at[srcv.at[b]], rows[b], gsem.at[b])
        cz.wait()
        pltpu.sync_copy(stage, acc.at[pl.ds(sid * orows, orows)])
        cd.wait()
        plsc.subcore_barrier()

        @pl.loop(0, nring)
        def _(j):
            base = j * nb
            for b in range(nb):
                i = base + b
                pltpu.make_async_copy(
                    g_hbm.at[srcv.at[i]], rows[b], gsem.at[b]).wait()
                pltpu.async_copy(rows[b], acc.at[dstv.at[i]], ssem.at[b],
                                 add=True)

            @pl.when(j < nring - 1)
            def _():
                for b in range(nb):
                    i = base + b
                    pltpu.make_async_copy(
                        rows[b], acc.at[dstv.at[i]], ssem.at[b]).wait()
                    pltpu.async_copy(
                        g_hbm.at[srcv.at[i + nb]], rows[b], gsem.at[b])

        for b in range(nb):
            i = (nring - 1) * nb + b
            pltpu.make_async_copy(
                rows[b], acc.at[dstv.at[i]], ssem.at[b]).wait()

        if tail:
            @pl.when(wid >= NW - tail)
            def _():
                tcid = n_chunks - NW + wid
                pltpu.sync_copy(e_hbm.at[0, tcid], srcv.at[0])
                pltpu.sync_copy(e_hbm.at[1, tcid], dstv.at[0])
                pltpu.sync_copy(g_hbm.at[srcv.at[0]], rows[0])
                pltpu.sync_copy(rows[0], acc.at[dstv.at[0]], add=True)

        plsc.subcore_barrier()
        pltpu.sync_copy(acc.at[pl.ds(sid * orows, orows)],
                        out_hbm.at[cid, pl.ds(sid * orows, orows)])

    return k(g, edge3d, zeros_tile)


def _tc_call(body, out_shapes, *args):
    return pl.pallas_call(
        body,
        out_shape=[jax.ShapeDtypeStruct(s, jnp.float32) for s in out_shapes],
    )(*args)


def _mm_body(x_ref, w_ref, o_ref):
    o_ref[...] = jnp.dot(x_ref[...], w_ref[...],
                         preferred_element_type=jnp.float32)


def _norm_body(degp_ref, h_ref, b_ref, g_ref, disb_ref, a_ref):
    m = h_ref.shape[0]
    degp = degp_ref[...]
    deg = degp[0, :m] + degp[1, :m] + 1.0
    disb = 1.0 / jnp.sqrt(deg)
    h = h_ref[...]
    g_ref[...] = disb * h
    disb_ref[...] = disb
    a_ref[...] = disb * disb * h + b_ref[...]


def _mid_body(sp_ref, disb_ref, a_ref, w2_ref, b2_ref, g2_ref, a2_ref):
    m = disb_ref.shape[0]
    disb = disb_ref[...]
    sp = sp_ref[...]
    out1 = jnp.maximum(disb * (sp[0, :m] + sp[1, :m]) + a_ref[...], 0.0)
    h2 = jnp.dot(out1, w2_ref[...], preferred_element_type=jnp.float32)
    g2_ref[...] = disb * h2
    a2_ref[...] = disb * disb * h2 + b2_ref[...]


def _final_body(sp_ref, disb_ref, a2_ref, o_ref):
    m = disb_ref.shape[0]
    sp = sp_ref[...]
    out2 = disb_ref[...] * (sp[0, :m] + sp[1, :m]) + a2_ref[...]
    cols = []
    for j in range(8):
        v = out2[:, 16 * j:16 * (j + 1)]
        z = v - jnp.max(v, axis=1, keepdims=True)
        cols.append(z - jnp.log(jnp.sum(jnp.exp(z), axis=1, keepdims=True)))
    o_ref[...] = jnp.concatenate(cols, axis=1)


@jax.jit
def _gcn(x, edge_index, W1, b1, W2, b2):
    n = x.shape[0]
    e = edge_index.shape[1]

    n_acc = -(-n // (NS * 8)) * NS * 8
    mp, ma = n // 8, n_acc // 8
    edge3d = edge_index.reshape(2, e // CHUNK, CHUNK)
    zeros_tile = jnp.zeros((n_acc // NS, LANES), jnp.float32)
    ones_chunk = jnp.ones((CHUNK, LANES), jnp.float32)
    b1p = jnp.tile(b1, 8).reshape(1, 128)
    b2p = jnp.tile(b2, 8).reshape(1, 128)
    eye8 = jnp.eye(8, dtype=jnp.float32)
    w1blk = jnp.reshape(
        eye8[:, None, :, None] * W1[None, :, None, :], (8 * 128, 128))
    w2blk = jnp.reshape(
        eye8[:, None, :, None] * W2[None, :, None, :], (128, 128))

    degp = _sc_degree(edge3d, zeros_tile, ones_chunk, n_acc)
    (h1p,) = _tc_call(_mm_body, [(mp, 128)], x.reshape(mp, 8 * 128), w1blk)
    g1p, disbp, a1p = _tc_call(_norm_body, [(mp, 128)] * 3,
                               degp.reshape(NC, ma, 128), h1p, b1p)
    s1p = _sc_gather_scatter(g1p.reshape(n, LANES), edge3d, zeros_tile, n_acc)
    g2p, a2p = _tc_call(_mid_body, [(mp, 128)] * 2,
                        s1p.reshape(NC, ma, 128), disbp, a1p, w2blk, b2p)
    s2p = _sc_gather_scatter(g2p.reshape(n, LANES), edge3d, zeros_tile, n_acc)
    (outp,) = _tc_call(_final_body, [(mp, 128)],
                       s2p.reshape(NC, ma, 128), disbp, a2p)
    return outp.reshape(n, LANES)


def kernel(x, edge_index, W1, b1, W2, b2):
    return _gcn(x, edge_index, W1, b1, W2, b2)
